# Initial kernel scaffold; baseline (speedup 1.0000x reference)
import jax
import jax.numpy as jnp
from jax.experimental import pallas as pl


def kernel(node_table, node_indices):
    raise NotImplementedError("write your pallas kernel here")


# speedup vs baseline: 2.4814x; 2.4814x over previous
import jax
import jax.numpy as jnp
from jax import lax
from jax.experimental import pallas as pl
from jax.experimental.pallas import tpu as pltpu
from jax.experimental.pallas import tpu_sc as plsc

VOCAB = 2048
DIM = 96
N_IDX = 16384
NC = 2
NS = 16
L = 16
NW = NC * NS
IDX_CHUNK = 128
CHUNKS = N_IDX // NW // IDX_CHUNK
D_VECS = DIM // L


def _body(table_hbm, idx_hbm, out_hbm,
          idx_v, rows_a, rows_b, acc_v, partials_v, out_v,
          partials_sh, sem_a, sem_b):
    c = lax.axis_index("c")
    s = lax.axis_index("s")
    w = s * NC + c

    pltpu.sync_copy(idx_hbm.at[pl.ds(w * CHUNKS, CHUNKS)], idx_v)

    rows = (rows_a, rows_b)
    sems = (sem_a, sem_b)
    cps = [None, None]
    cps[0] = pltpu.async_copy(table_hbm.at[idx_v.at[0]], rows_a, sem_a)

    def chunk_sum(buf):
        def row_fma(v, acc):
            return tuple(acc[d] + buf[v, pl.ds(d * L, L)]
                         for d in range(D_VECS))
        return row_fma

    acc = (jnp.zeros((L,), jnp.float32),) * D_VECS
    for j in range(CHUNKS):
        if j + 1 < CHUNKS:
            nxt = (j + 1) % 2
            cps[nxt] = pltpu.async_copy(
                table_hbm.at[idx_v.at[j + 1]], rows[nxt], sems[nxt])
        cps[j % 2].wait()
        acc = lax.fori_loop(0, IDX_CHUNK, chunk_sum(rows[j % 2]), acc,
                            unroll=8)

    for d in range(D_VECS):
        acc_v[pl.ds(d * L, L)] = acc[d]
    pltpu.sync_copy(acc_v, partials_sh.at[s])
    plsc.subcore_barrier()

    @pl.when(s == 0)
    def _():
        pltpu.sync_copy(partials_sh, partials_v)
        for d in range(D_VECS):
            tot = jnp.zeros((L,), jnp.float32)
            for tt in range(NS):
                tot = tot + partials_v[tt, pl.ds(d * L, L)]
            out_v[pl.ds(d * L, L)] = tot * (1.0 / N_IDX)
        pltpu.sync_copy(out_v, out_hbm.at[c])


@jax.jit
def _run(table, idx):
    mesh = plsc.VectorSubcoreMesh(
        core_axis_name="c", subcore_axis_name="s", num_cores=NC)
    f = pl.kernel(
        _body,
        out_type=jax.ShapeDtypeStruct((NC, DIM), jnp.float32),
        mesh=mesh,
        compiler_params=pltpu.CompilerParams(use_tc_tiling_on_sc=False),
        scratch_types=[
            pltpu.VMEM((CHUNKS, IDX_CHUNK), jnp.int32),
            pltpu.VMEM((IDX_CHUNK, DIM), jnp.float32),
            pltpu.VMEM((IDX_CHUNK, DIM), jnp.float32),
            pltpu.VMEM((DIM,), jnp.float32),
            pltpu.VMEM((NS, DIM), jnp.float32),
            pltpu.VMEM((DIM,), jnp.float32),
            pltpu.VMEM_SHARED((NS, DIM), jnp.float32),
            pltpu.SemaphoreType.DMA,
            pltpu.SemaphoreType.DMA,
        ],
    )
    partial = f(table, idx)
    return partial.sum(axis=0)


def kernel(node_table, node_indices):
    idx = node_indices.astype(jnp.int32).reshape(NW * CHUNKS, IDX_CHUNK)
    return _run(node_table, idx)


# speedup vs baseline: 2.4859x; 1.0018x over previous
import jax
import jax.numpy as jnp
from jax import lax
from jax.experimental import pallas as pl
from jax.experimental.pallas import tpu as pltpu
from jax.experimental.pallas import tpu_sc as plsc

VOCAB = 2048
DIM = 96
N_IDX = 16384
NC = 2
NS = 16
L = 16
NW = NC * NS
IDX_CHUNK = 128
CHUNKS = N_IDX // NW // IDX_CHUNK
PER_W = IDX_CHUNK * CHUNKS
D_VECS = DIM // L


def _body(table_hbm, idx_hbm, out_hbm,
          idx_v, rows_v, acc_v, partials_v,
          partials_sh, sems):
    c = lax.axis_index("c")
    s = lax.axis_index("s")
    w = s * NC + c

    pltpu.sync_copy(idx_hbm.at[pl.ds(w * PER_W, PER_W)], idx_v)

    cps = [
        pltpu.async_copy(
            table_hbm.at[idx_v.at[pl.ds(j * IDX_CHUNK, IDX_CHUNK)]],
            rows_v.at[pl.ds(j * IDX_CHUNK, IDX_CHUNK)],
            sems.at[j])
        for j in range(CHUNKS)
    ]
    for cp in cps:
        cp.wait()

    def row_sum(v, acc):
        return tuple(acc[d] + rows_v[v, pl.ds(d * L, L)]
                     for d in range(D_VECS))

    acc = lax.fori_loop(0, PER_W, row_sum,
                        (jnp.zeros((L,), jnp.float32),) * D_VECS,
                        unroll=4)
    for d in range(D_VECS):
        acc_v[pl.ds(d * L, L)] = acc[d]
    pltpu.sync_copy(acc_v, partials_sh.at[s])
    plsc.subcore_barrier()

    @pl.when(s == 0)
    def _():
        pltpu.sync_copy(partials_sh, partials_v)

        def tile_sum(tt, acc):
            return tuple(acc[d] + partials_v[tt, pl.ds(d * L, L)]
                         for d in range(D_VECS))

        tot = lax.fori_loop(0, NS, tile_sum,
                            (jnp.zeros((L,), jnp.float32),) * D_VECS)
        for d in range(D_VECS):
            acc_v[pl.ds(d * L, L)] = tot[d] * (1.0 / N_IDX)
        pltpu.sync_copy(acc_v, out_hbm.at[c])


@jax.jit
def _run(table, idx):
    mesh = plsc.VectorSubcoreMesh(
        core_axis_name="c", subcore_axis_name="s", num_cores=NC)
    f = pl.kernel(
        _body,
        out_type=jax.ShapeDtypeStruct((NC, DIM), jnp.float32),
        mesh=mesh,
        compiler_params=pltpu.CompilerParams(use_tc_tiling_on_sc=False),
        scratch_types=[
            pltpu.VMEM((PER_W,), jnp.int32),
            pltpu.VMEM((PER_W, DIM), jnp.float32),
            pltpu.VMEM((DIM,), jnp.float32),
            pltpu.VMEM((NS, DIM), jnp.float32),
            pltpu.VMEM_SHARED((NS, DIM), jnp.float32),
            pltpu.SemaphoreType.DMA((CHUNKS,)),
        ],
    )
    partial = f(table, idx)
    return partial.sum(axis=0)


def kernel(node_table, node_indices):
    return _run(node_table, node_indices.astype(jnp.int32))


# speedup vs baseline: 2.4878x; 1.0007x over previous
import jax
import jax.numpy as jnp
from jax import lax
from jax.experimental import pallas as pl
from jax.experimental.pallas import tpu as pltpu
from jax.experimental.pallas import tpu_sc as plsc

VOCAB = 2048
DIM = 96
N_IDX = 16384
NC = 2
NS = 16
L = 16
NW = NC * NS
IDX_CHUNK = 128
CHUNKS = N_IDX // NW // IDX_CHUNK
PER_W = IDX_CHUNK * CHUNKS
D_VECS = DIM // L


def _body(table_hbm, idx_hbm, out_hbm,
          idx_v, rows_v, acc_v, partials_v,
          partials_sh, sems):
    c = lax.axis_index("c")
    s = lax.axis_index("s")
    w = s * NC + c

    pltpu.sync_copy(idx_hbm.at[pl.ds(w * PER_W, PER_W)], idx_v)

    cps = [
        pltpu.async_copy(
            table_hbm.at[idx_v.at[pl.ds(j * IDX_CHUNK, IDX_CHUNK)]],
            rows_v.at[pl.ds(j * IDX_CHUNK, IDX_CHUNK)],
            sems.at[j])
        for j in range(CHUNKS)
    ]
    for cp in cps:
        cp.wait()

    def row_sum(v, acc):
        return tuple(acc[d] + rows_v[v, pl.ds(d * L, L)]
                     for d in range(D_VECS))

    acc = lax.fori_loop(0, PER_W, row_sum,
                        (jnp.zeros((L,), jnp.float32),) * D_VECS,
                        unroll=4)
    for d in range(D_VECS):
        acc_v[pl.ds(d * L, L)] = acc[d]
    pltpu.sync_copy(acc_v, partials_sh.at[s])
    plsc.subcore_barrier()

    @pl.when(s == 0)
    def _():
        pltpu.sync_copy(partials_sh, partials_v)

        def tile_sum(tt, acc):
            return tuple(acc[d] + partials_v[tt, pl.ds(d * L, L)]
                         for d in range(D_VECS))

        tot = lax.fori_loop(0, NS, tile_sum,
                            (jnp.zeros((L,), jnp.float32),) * D_VECS)
        for d in range(D_VECS):
            acc_v[pl.ds(d * L, L)] = tot[d] * (1.0 / N_IDX)
        pltpu.sync_copy(acc_v, out_hbm.at[c])


@jax.jit
def _run(table, idx):
    mesh = plsc.VectorSubcoreMesh(
        core_axis_name="c", subcore_axis_name="s", num_cores=NC)
    f = pl.kernel(
        _body,
        out_type=jax.ShapeDtypeStruct((NC, DIM), jnp.float32),
        mesh=mesh,
        compiler_params=pltpu.CompilerParams(
            use_tc_tiling_on_sc=False, skip_device_barrier=True),
        scratch_types=[
            pltpu.VMEM((PER_W,), jnp.int32),
            pltpu.VMEM((PER_W, DIM), jnp.float32),
            pltpu.VMEM((DIM,), jnp.float32),
            pltpu.VMEM((NS, DIM), jnp.float32),
            pltpu.VMEM_SHARED((NS, DIM), jnp.float32),
            pltpu.SemaphoreType.DMA((CHUNKS,)),
        ],
    )
    partial = f(table, idx)
    return partial.sum(axis=0)


def kernel(node_table, node_indices):
    return _run(node_table, node_indices.astype(jnp.int32))


# speedup vs baseline: 2.6886x; 1.0807x over previous
import jax
import jax.numpy as jnp
from jax import lax
from jax.experimental import pallas as pl
from jax.experimental.pallas import tpu as pltpu
from jax.experimental.pallas import tpu_sc as plsc

VOCAB = 2048
DIM = 96
N_IDX = 16384
NC = 2
NS = 16
L = 16
NW = NC * NS
IDX_CHUNK = 128
CHUNKS = N_IDX // NW // IDX_CHUNK
PER_W = IDX_CHUNK * CHUNKS
V_PER_T = VOCAB // NS
V_VECS = V_PER_T // L
D_VECS = DIM // L

def _body(table_hbm, idx_hbm, out_hbm,
          tabrows_v, idx_v, idx2_v, ones_v, hist_v, merge_v, counts_v,
          acc_v, partials_v,
          hists_sh, partials_sh, tab_sem, idx_sem):
    c = lax.axis_index("c")
    s = lax.axis_index("s")
    w = s * NC + c

    tab_cp = pltpu.async_copy(
        table_hbm.at[pl.ds(s * V_PER_T, V_PER_T)], tabrows_v, tab_sem)

    idx_cps = [
        pltpu.async_copy(
            idx_hbm.at[pl.ds(w * PER_W + j * IDX_CHUNK, IDX_CHUNK)],
            idx_v.at[j], idx_sem)
        for j in range(CHUNKS)
    ]

    for i in range(IDX_CHUNK // L):
        ones_v[pl.ds(i * L, L)] = jnp.ones((L,), jnp.float32)

    def zero_hist(i, carry):
        hist_v[pl.ds(i * L, L)] = jnp.zeros((L,), jnp.float32)
        return carry

    lax.fori_loop(0, VOCAB // L, zero_hist, 0, unroll=8)
    pltpu.sync_copy(hist_v, hists_sh.at[pl.ds(s * VOCAB, VOCAB)])
    for cp in idx_cps:
        cp.wait()

    off = s * VOCAB
    for j in range(CHUNKS):
        for i in range(IDX_CHUNK // L):
            idx2_v[j, pl.ds(i * L, L)] = idx_v[j, pl.ds(i * L, L)] + off

    for j in range(CHUNKS):
        pltpu.sync_copy(ones_v, hists_sh.at[idx2_v.at[j]], add=True)

    plsc.subcore_barrier()
    for i in range(NS):
        pltpu.sync_copy(hists_sh.at[pl.ds(i * VOCAB + s * V_PER_T, V_PER_T)],
                        merge_v.at[i])

    def merge_tile(i, acc):
        return tuple(acc[v] + merge_v[i, pl.ds(v * L, L)]
                     for v in range(V_VECS))

    csum = lax.fori_loop(0, NS, merge_tile,
                         (jnp.zeros((L,), jnp.float32),) * V_VECS)
    for v in range(V_VECS):
        counts_v[pl.ds(v * L, L)] = csum[v]

    tab_cp.wait()

    def fma_group(g, acc):
        cvec = counts_v[pl.ds(g * L, L)]
        for k in range(L):
            cb = lax.gather(
                cvec, jnp.full((L, 1), k, jnp.int32),
                dimension_numbers=lax.GatherDimensionNumbers(
                    offset_dims=(), collapsed_slice_dims=(0,),
                    start_index_map=(0,)),
                slice_sizes=(1,),
                mode=lax.GatherScatterMode.PROMISE_IN_BOUNDS)
            acc = tuple(acc[d] + cb * tabrows_v[g * L + k, pl.ds(d * L, L)]
                        for d in range(D_VECS))
        return acc

    acc = lax.fori_loop(0, V_VECS, fma_group,
                        (jnp.zeros((L,), jnp.float32),) * D_VECS)
    for d in range(D_VECS):
        acc_v[pl.ds(d * L, L)] = acc[d]
    pltpu.sync_copy(acc_v, partials_sh.at[s])
    plsc.subcore_barrier()

    @pl.when(s == 0)
    def _():
        pltpu.sync_copy(partials_sh, partials_v)

        def tile_sum(tt, acc):
            return tuple(acc[d] + partials_v[tt, pl.ds(d * L, L)]
                         for d in range(D_VECS))

        tot = lax.fori_loop(0, NS, tile_sum,
                            (jnp.zeros((L,), jnp.float32),) * D_VECS)
        for d in range(D_VECS):
            acc_v[pl.ds(d * L, L)] = tot[d] * (1.0 / N_IDX)
        pltpu.sync_copy(acc_v, out_hbm.at[c])


@jax.jit
def _run(table, idx):
    mesh = plsc.VectorSubcoreMesh(
        core_axis_name="c", subcore_axis_name="s", num_cores=NC)
    f = pl.kernel(
        _body,
        out_type=jax.ShapeDtypeStruct((NC, DIM), jnp.float32),
        mesh=mesh,
        compiler_params=pltpu.CompilerParams(
            use_tc_tiling_on_sc=False, skip_device_barrier=True),
        scratch_types=[
            pltpu.VMEM((V_PER_T, DIM), jnp.float32),
            pltpu.VMEM((CHUNKS, IDX_CHUNK), jnp.int32),
            pltpu.VMEM((CHUNKS, IDX_CHUNK), jnp.int32),
            pltpu.VMEM((IDX_CHUNK,), jnp.float32),
            pltpu.VMEM((VOCAB,), jnp.float32),
            pltpu.VMEM((NS, V_PER_T), jnp.float32),
            pltpu.VMEM((V_PER_T,), jnp.float32),
            pltpu.VMEM((DIM,), jnp.float32),
            pltpu.VMEM((NS, DIM), jnp.float32),
            pltpu.VMEM_SHARED((NS * VOCAB,), jnp.float32),
            pltpu.VMEM_SHARED((NS, DIM), jnp.float32),
            pltpu.SemaphoreType.DMA,
            pltpu.SemaphoreType.DMA,
        ],
    )
    partial = f(table, idx)
    return partial.sum(axis=0)


def kernel(node_table, node_indices):
    return _run(node_table, node_indices.astype(jnp.int32))


# speedup vs baseline: 2.9646x; 1.1026x over previous
import jax
import jax.numpy as jnp
from jax import lax
from jax.experimental import pallas as pl
from jax.experimental.pallas import tpu as pltpu
from jax.experimental.pallas import tpu_sc as plsc

VOCAB = 2048
DIM = 96
N_IDX = 16384
NC = 1
NS = 16
L = 16
NW = NC * NS
IDX_CHUNK = 128
CHUNKS = N_IDX // NW // IDX_CHUNK
PER_W = IDX_CHUNK * CHUNKS
V_PER_T = VOCAB // NS
V_VECS = V_PER_T // L
D_VECS = DIM // L

def _body(table_hbm, idx_hbm, out_hbm,
          tabrows_v, idx_v, idx2_v, ones_v, hist_v, merge_v, counts_v,
          acc_v, partials_v,
          hists_sh, partials_sh, tab_sem, idx_sem):
    s = lax.axis_index("s")
    w = s

    tab_cp = pltpu.async_copy(
        table_hbm.at[pl.ds(s * V_PER_T, V_PER_T)], tabrows_v, tab_sem)

    idx_cps = [
        pltpu.async_copy(
            idx_hbm.at[pl.ds(w * PER_W + j * IDX_CHUNK, IDX_CHUNK)],
            idx_v.at[j], idx_sem)
        for j in range(CHUNKS)
    ]

    for i in range(IDX_CHUNK // L):
        ones_v[pl.ds(i * L, L)] = jnp.ones((L,), jnp.float32)

    def zero_hist(i, carry):
        hist_v[pl.ds(i * L, L)] = jnp.zeros((L,), jnp.float32)
        return carry

    lax.fori_loop(0, VOCAB // L, zero_hist, 0, unroll=8)
    pltpu.sync_copy(hist_v, hists_sh.at[pl.ds(s * VOCAB, VOCAB)])
    for cp in idx_cps:
        cp.wait()

    off = s * VOCAB
    for j in range(CHUNKS):
        for i in range(IDX_CHUNK // L):
            idx2_v[j, pl.ds(i * L, L)] = idx_v[j, pl.ds(i * L, L)] + off

    for j in range(CHUNKS):
        pltpu.sync_copy(ones_v, hists_sh.at[idx2_v.at[j]], add=True)

    plsc.subcore_barrier()
    for i in range(NS):
        pltpu.sync_copy(hists_sh.at[pl.ds(i * VOCAB + s * V_PER_T, V_PER_T)],
                        merge_v.at[i])

    def merge_tile(i, acc):
        return tuple(acc[v] + merge_v[i, pl.ds(v * L, L)]
                     for v in range(V_VECS))

    csum = lax.fori_loop(0, NS, merge_tile,
                         (jnp.zeros((L,), jnp.float32),) * V_VECS)
    for v in range(V_VECS):
        counts_v[pl.ds(v * L, L)] = csum[v]

    tab_cp.wait()

    def fma_group(g, acc):
        cvec = counts_v[pl.ds(g * L, L)]
        for k in range(L):
            cb = lax.gather(
                cvec, jnp.full((L, 1), k, jnp.int32),
                dimension_numbers=lax.GatherDimensionNumbers(
                    offset_dims=(), collapsed_slice_dims=(0,),
                    start_index_map=(0,)),
                slice_sizes=(1,),
                mode=lax.GatherScatterMode.PROMISE_IN_BOUNDS)
            acc = tuple(acc[d] + cb * tabrows_v[g * L + k, pl.ds(d * L, L)]
                        for d in range(D_VECS))
        return acc

    acc = lax.fori_loop(0, V_VECS, fma_group,
                        (jnp.zeros((L,), jnp.float32),) * D_VECS)
    for d in range(D_VECS):
        acc_v[pl.ds(d * L, L)] = acc[d]
    pltpu.sync_copy(acc_v, partials_sh.at[s])
    plsc.subcore_barrier()

    @pl.when(s == 0)
    def _():
        pltpu.sync_copy(partials_sh, partials_v)

        def tile_sum(tt, acc):
            return tuple(acc[d] + partials_v[tt, pl.ds(d * L, L)]
                         for d in range(D_VECS))

        tot = lax.fori_loop(0, NS, tile_sum,
                            (jnp.zeros((L,), jnp.float32),) * D_VECS)
        for d in range(D_VECS):
            acc_v[pl.ds(d * L, L)] = tot[d] * (1.0 / N_IDX)
        pltpu.sync_copy(acc_v, out_hbm)


@jax.jit
def _run(table, idx):
    mesh = plsc.VectorSubcoreMesh(
        core_axis_name="c", subcore_axis_name="s", num_cores=NC)
    f = pl.kernel(
        _body,
        out_type=jax.ShapeDtypeStruct((DIM,), jnp.float32),
        mesh=mesh,
        compiler_params=pltpu.CompilerParams(
            use_tc_tiling_on_sc=False, skip_device_barrier=True),
        scratch_types=[
            pltpu.VMEM((V_PER_T, DIM), jnp.float32),
            pltpu.VMEM((CHUNKS, IDX_CHUNK), jnp.int32),
            pltpu.VMEM((CHUNKS, IDX_CHUNK), jnp.int32),
            pltpu.VMEM((IDX_CHUNK,), jnp.float32),
            pltpu.VMEM((VOCAB,), jnp.float32),
            pltpu.VMEM((NS, V_PER_T), jnp.float32),
            pltpu.VMEM((V_PER_T,), jnp.float32),
            pltpu.VMEM((DIM,), jnp.float32),
            pltpu.VMEM((NS, DIM), jnp.float32),
            pltpu.VMEM_SHARED((NS * VOCAB,), jnp.float32),
            pltpu.VMEM_SHARED((NS, DIM), jnp.float32),
            pltpu.SemaphoreType.DMA,
            pltpu.SemaphoreType.DMA,
        ],
    )
    return f(table, idx)


def kernel(node_table, node_indices):
    return _run(node_table, node_indices.astype(jnp.int32))


# speedup vs baseline: 3.2178x; 1.0854x over previous
import jax
import jax.numpy as jnp
from jax import lax
from jax.experimental import pallas as pl
from jax.experimental.pallas import tpu as pltpu
from jax.experimental.pallas import tpu_sc as plsc

VOCAB = 2048
DIM = 96
N_IDX = 16384
NC = 1
NS = 16
L = 16
NW = NC * NS
IDX_CHUNK = 128
CHUNKS = N_IDX // NW // IDX_CHUNK
PER_W = IDX_CHUNK * CHUNKS
V_PER_T = VOCAB // NS
V_VECS = V_PER_T // L
D_VECS = DIM // L

def _body(table_hbm, idx_hbm, out_hbm,
          tabrows_v, idx_v, ones_v, hist_v, counts_v,
          acc_v, partials_v,
          hist_sh, partials_sh, tab_sem, idx_sem):
    s = lax.axis_index("s")
    w = s

    tab_cp = pltpu.async_copy(
        table_hbm.at[pl.ds(s * V_PER_T, V_PER_T)], tabrows_v, tab_sem)

    idx_cps = [
        pltpu.async_copy(
            idx_hbm.at[pl.ds(w * PER_W + j * IDX_CHUNK, IDX_CHUNK)],
            idx_v.at[j], idx_sem)
        for j in range(CHUNKS)
    ]

    for i in range(IDX_CHUNK // L):
        ones_v[pl.ds(i * L, L)] = jnp.ones((L,), jnp.float32)

    @pl.when(s == 0)
    def _():
        def zero_hist(i, carry):
            hist_v[pl.ds(i * L, L)] = jnp.zeros((L,), jnp.float32)
            return carry

        lax.fori_loop(0, VOCAB // L, zero_hist, 0, unroll=8)
        pltpu.sync_copy(hist_v, hist_sh)

    for cp in idx_cps:
        cp.wait()
    plsc.subcore_barrier()

    for j in range(CHUNKS):
        pltpu.sync_copy(ones_v, hist_sh.at[idx_v.at[j]], add=True)

    plsc.subcore_barrier()
    pltpu.sync_copy(hist_sh.at[pl.ds(s * V_PER_T, V_PER_T)], counts_v)

    tab_cp.wait()

    def fma_group(g, acc):
        cvec = counts_v[pl.ds(g * L, L)]
        for k in range(L):
            cb = lax.gather(
                cvec, jnp.full((L, 1), k, jnp.int32),
                dimension_numbers=lax.GatherDimensionNumbers(
                    offset_dims=(), collapsed_slice_dims=(0,),
                    start_index_map=(0,)),
                slice_sizes=(1,),
                mode=lax.GatherScatterMode.PROMISE_IN_BOUNDS)
            acc = tuple(acc[d] + cb * tabrows_v[g * L + k, pl.ds(d * L, L)]
                        for d in range(D_VECS))
        return acc

    acc = lax.fori_loop(0, V_VECS, fma_group,
                        (jnp.zeros((L,), jnp.float32),) * D_VECS)
    for d in range(D_VECS):
        acc_v[pl.ds(d * L, L)] = acc[d]
    pltpu.sync_copy(acc_v, partials_sh.at[s])
    plsc.subcore_barrier()

    @pl.when(s == 0)
    def _():
        pltpu.sync_copy(partials_sh, partials_v)

        def tile_sum(tt, acc):
            return tuple(acc[d] + partials_v[tt, pl.ds(d * L, L)]
                         for d in range(D_VECS))

        tot = lax.fori_loop(0, NS, tile_sum,
                            (jnp.zeros((L,), jnp.float32),) * D_VECS)
        for d in range(D_VECS):
            acc_v[pl.ds(d * L, L)] = tot[d] * (1.0 / N_IDX)
        pltpu.sync_copy(acc_v, out_hbm)


@jax.jit
def _run(table, idx):
    mesh = plsc.VectorSubcoreMesh(
        core_axis_name="c", subcore_axis_name="s", num_cores=NC)
    f = pl.kernel(
        _body,
        out_type=jax.ShapeDtypeStruct((DIM,), jnp.float32),
        mesh=mesh,
        compiler_params=pltpu.CompilerParams(
            use_tc_tiling_on_sc=False, skip_device_barrier=True),
        scratch_types=[
            pltpu.VMEM((V_PER_T, DIM), jnp.float32),
            pltpu.VMEM((CHUNKS, IDX_CHUNK), jnp.int32),
            pltpu.VMEM((IDX_CHUNK,), jnp.float32),
            pltpu.VMEM((VOCAB,), jnp.float32),
            pltpu.VMEM((V_PER_T,), jnp.float32),
            pltpu.VMEM((DIM,), jnp.float32),
            pltpu.VMEM((NS, DIM), jnp.float32),
            pltpu.VMEM_SHARED((VOCAB,), jnp.float32),
            pltpu.VMEM_SHARED((NS, DIM), jnp.float32),
            pltpu.SemaphoreType.DMA,
            pltpu.SemaphoreType.DMA,
        ],
    )
    return f(table, idx)


def kernel(node_table, node_indices):
    return _run(node_table, node_indices.astype(jnp.int32))


# speedup vs baseline: 3.2809x; 1.0196x over previous
import jax
import jax.numpy as jnp
from jax import lax
from jax.experimental import pallas as pl
from jax.experimental.pallas import tpu as pltpu
from jax.experimental.pallas import tpu_sc as plsc

VOCAB = 2048
DIM = 96
N_IDX = 16384
NC = 1
NS = 16
L = 16
NW = NC * NS
IDX_CHUNK = 128
CHUNKS = N_IDX // NW // IDX_CHUNK
PER_W = IDX_CHUNK * CHUNKS
V_PER_T = VOCAB // NS
V_VECS = V_PER_T // L
D_VECS = DIM // L

def _body(table_hbm, idx_hbm, out_hbm,
          tabrows_v, idx_v, ones_v, hist_v, counts_v,
          acc_v, partials_v,
          hist_sh, partials_sh, tab_sem, idx_sem):
    s = lax.axis_index("s")
    w = s

    tab_cp = pltpu.async_copy(
        table_hbm.at[pl.ds(s * V_PER_T, V_PER_T)], tabrows_v, tab_sem)

    idx_cps = [
        pltpu.async_copy(
            idx_hbm.at[pl.ds(w * PER_W + j * IDX_CHUNK, IDX_CHUNK)],
            idx_v.at[j], idx_sem)
        for j in range(CHUNKS)
    ]

    for i in range(IDX_CHUNK // L):
        ones_v[pl.ds(i * L, L)] = jnp.ones((L,), jnp.float32)

    @pl.when(s == 0)
    def _():
        def zero_hist(i, carry):
            hist_v[pl.ds(i * L, L)] = jnp.zeros((L,), jnp.float32)
            return carry

        lax.fori_loop(0, VOCAB // L, zero_hist, 0, unroll=8)
        pltpu.sync_copy(hist_v, hist_sh)

    for cp in idx_cps:
        cp.wait()
    plsc.subcore_barrier()

    sc_cps = [
        pltpu.async_copy(ones_v, hist_sh.at[idx_v.at[j]], idx_sem, add=True)
        for j in range(CHUNKS)
    ]
    for cp in sc_cps:
        cp.wait()

    plsc.subcore_barrier()
    pltpu.sync_copy(hist_sh.at[pl.ds(s * V_PER_T, V_PER_T)], counts_v)

    tab_cp.wait()

    def fma_group(g, acc):
        cvec = counts_v[pl.ds(g * L, L)]
        for k in range(L):
            cb = lax.gather(
                cvec, jnp.full((L, 1), k, jnp.int32),
                dimension_numbers=lax.GatherDimensionNumbers(
                    offset_dims=(), collapsed_slice_dims=(0,),
                    start_index_map=(0,)),
                slice_sizes=(1,),
                mode=lax.GatherScatterMode.PROMISE_IN_BOUNDS)
            acc = tuple(acc[d] + cb * tabrows_v[g * L + k, pl.ds(d * L, L)]
                        for d in range(D_VECS))
        return acc

    acc = lax.fori_loop(0, V_VECS, fma_group,
                        (jnp.zeros((L,), jnp.float32),) * D_VECS)
    for d in range(D_VECS):
        acc_v[pl.ds(d * L, L)] = acc[d]
    pltpu.sync_copy(acc_v, partials_sh.at[s])
    plsc.subcore_barrier()

    @pl.when(s == 0)
    def _():
        pltpu.sync_copy(partials_sh, partials_v)

        def tile_sum(tt, acc):
            return tuple(acc[d] + partials_v[tt, pl.ds(d * L, L)]
                         for d in range(D_VECS))

        tot = lax.fori_loop(0, NS, tile_sum,
                            (jnp.zeros((L,), jnp.float32),) * D_VECS)
        for d in range(D_VECS):
            acc_v[pl.ds(d * L, L)] = tot[d] * (1.0 / N_IDX)
        pltpu.sync_copy(acc_v, out_hbm)


@jax.jit
def _run(table, idx):
    mesh = plsc.VectorSubcoreMesh(
        core_axis_name="c", subcore_axis_name="s", num_cores=NC)
    f = pl.kernel(
        _body,
        out_type=jax.ShapeDtypeStruct((DIM,), jnp.float32),
        mesh=mesh,
        compiler_params=pltpu.CompilerParams(
            use_tc_tiling_on_sc=False, skip_device_barrier=True),
        scratch_types=[
            pltpu.VMEM((V_PER_T, DIM), jnp.float32),
            pltpu.VMEM((CHUNKS, IDX_CHUNK), jnp.int32),
            pltpu.VMEM((IDX_CHUNK,), jnp.float32),
            pltpu.VMEM((VOCAB,), jnp.float32),
            pltpu.VMEM((V_PER_T,), jnp.float32),
            pltpu.VMEM((DIM,), jnp.float32),
            pltpu.VMEM((NS, DIM), jnp.float32),
            pltpu.VMEM_SHARED((VOCAB,), jnp.float32),
            pltpu.VMEM_SHARED((NS, DIM), jnp.float32),
            pltpu.SemaphoreType.DMA,
            pltpu.SemaphoreType.DMA,
        ],
    )
    return f(table, idx)


def kernel(node_table, node_indices):
    return _run(node_table, node_indices.astype(jnp.int32))


# speedup vs baseline: 3.2969x; 1.0049x over previous
import jax
import jax.numpy as jnp
from jax import lax
from jax.experimental import pallas as pl
from jax.experimental.pallas import tpu as pltpu
from jax.experimental.pallas import tpu_sc as plsc

VOCAB = 2048
DIM = 96
N_IDX = 16384
NC = 1
NS = 16
L = 16
NW = NC * NS
IDX_CHUNK = 128
CHUNKS = N_IDX // NW // IDX_CHUNK
PER_W = IDX_CHUNK * CHUNKS
V_PER_T = VOCAB // NS
V_VECS = V_PER_T // L
D_VECS = DIM // L

def _body(table_hbm, idx_hbm, out_hbm,
          tabrows_v, idx_v, ones_v, hist_v, counts_v,
          acc_v, partials_v,
          hist_sh, partials_sh, tab_sem, idx_sem):
    s = lax.axis_index("s")
    w = s

    tab_cp = pltpu.async_copy(
        table_hbm.at[pl.ds(s * V_PER_T, V_PER_T)], tabrows_v, tab_sem)

    idx_cps = [
        pltpu.async_copy(
            idx_hbm.at[pl.ds(w * PER_W + j * IDX_CHUNK, IDX_CHUNK)],
            idx_v.at[j], idx_sem)
        for j in range(CHUNKS)
    ]

    for i in range(IDX_CHUNK // L):
        ones_v[pl.ds(i * L, L)] = jnp.ones((L,), jnp.float32)
    for i in range(V_PER_T // L):
        hist_v[pl.ds(i * L, L)] = jnp.zeros((L,), jnp.float32)
    pltpu.sync_copy(hist_v, hist_sh.at[pl.ds(s * V_PER_T, V_PER_T)])
    plsc.subcore_barrier()
    for cp in idx_cps:
        cp.wait()

    sc_cps = [
        pltpu.async_copy(ones_v, hist_sh.at[idx_v.at[j]], idx_sem, add=True)
        for j in range(CHUNKS)
    ]
    for cp in sc_cps:
        cp.wait()

    plsc.subcore_barrier()
    pltpu.sync_copy(hist_sh.at[pl.ds(s * V_PER_T, V_PER_T)], counts_v)

    tab_cp.wait()

    def fma_group(g, acc):
        cvec = counts_v[pl.ds(g * L, L)]
        for k in range(L):
            cb = lax.gather(
                cvec, jnp.full((L, 1), k, jnp.int32),
                dimension_numbers=lax.GatherDimensionNumbers(
                    offset_dims=(), collapsed_slice_dims=(0,),
                    start_index_map=(0,)),
                slice_sizes=(1,),
                mode=lax.GatherScatterMode.PROMISE_IN_BOUNDS)
            acc = tuple(acc[d] + cb * tabrows_v[g * L + k, pl.ds(d * L, L)]
                        for d in range(D_VECS))
        return acc

    acc = lax.fori_loop(0, V_VECS, fma_group,
                        (jnp.zeros((L,), jnp.float32),) * D_VECS)
    for d in range(D_VECS):
        acc_v[pl.ds(d * L, L)] = acc[d]
    pltpu.sync_copy(acc_v, partials_sh.at[s])
    plsc.subcore_barrier()

    @pl.when(s == 0)
    def _():
        pltpu.sync_copy(partials_sh, partials_v)

        def tile_sum(tt, acc):
            return tuple(acc[d] + partials_v[tt, pl.ds(d * L, L)]
                         for d in range(D_VECS))

        tot = lax.fori_loop(0, NS, tile_sum,
                            (jnp.zeros((L,), jnp.float32),) * D_VECS)
        for d in range(D_VECS):
            acc_v[pl.ds(d * L, L)] = tot[d] * (1.0 / N_IDX)
        pltpu.sync_copy(acc_v, out_hbm)


@jax.jit
def _run(table, idx):
    mesh = plsc.VectorSubcoreMesh(
        core_axis_name="c", subcore_axis_name="s", num_cores=NC)
    f = pl.kernel(
        _body,
        out_type=jax.ShapeDtypeStruct((DIM,), jnp.float32),
        mesh=mesh,
        compiler_params=pltpu.CompilerParams(
            use_tc_tiling_on_sc=False, skip_device_barrier=True),
        scratch_types=[
            pltpu.VMEM((V_PER_T, DIM), jnp.float32),
            pltpu.VMEM((CHUNKS, IDX_CHUNK), jnp.int32),
            pltpu.VMEM((IDX_CHUNK,), jnp.float32),
            pltpu.VMEM((V_PER_T,), jnp.float32),
            pltpu.VMEM((V_PER_T,), jnp.float32),
            pltpu.VMEM((DIM,), jnp.float32),
            pltpu.VMEM((NS, DIM), jnp.float32),
            pltpu.VMEM_SHARED((VOCAB,), jnp.float32),
            pltpu.VMEM_SHARED((NS, DIM), jnp.float32),
            pltpu.SemaphoreType.DMA,
            pltpu.SemaphoreType.DMA,
        ],
    )
    return f(table, idx)


def kernel(node_table, node_indices):
    return _run(node_table, node_indices.astype(jnp.int32))
